# baseline (speedup 1.0000x reference)
import jax
import jax.numpy as jnp
from jax import lax
from jax.experimental import pallas as pl

_ALPHA = 0.95


def _row_sum_compensated(x):
    c = x.shape[1]
    partials = [
        jnp.sum(x[:, k : min(k + 128, c)], axis=1, keepdims=True)
        for k in range(0, c, 128)
    ]
    s = partials[0]
    comp = jnp.zeros_like(s)
    for p in partials[1:]:
        t = s + p
        comp = comp + jnp.where(
            jnp.abs(s) >= jnp.abs(p), (s - t) + p, (p - t) + s
        )
        s = t
    return s + comp


def _loca_body(x_ref, lab_ref, out_ref):
    x = x_ref[...]
    lab = lab_ref[...]
    r, c = x.shape
    col = lax.broadcasted_iota(jnp.int32, (r, c), 1)
    mask = col == lab
    rs = _row_sum_compensated(x)
    t = jnp.sum(jnp.where(mask, x, 0.0), axis=1, keepdims=True)
    s = _ALPHA / (1.0 - 2.0 * t + rs)
    tv = 1.0 - s * rs + s * t
    out_ref[...] = jnp.where(mask, tv, s * x)


def kernel(teacher_logits, true_labels):
    b, c = teacher_logits.shape
    rows = 512
    lab2 = true_labels.astype(jnp.int32).reshape(b, 1)
    return pl.pallas_call(
        _loca_body,
        grid=(b // rows,),
        in_specs=[
            pl.BlockSpec((rows, c), lambda i: (i, 0)),
            pl.BlockSpec((rows, 1), lambda i: (i, 0)),
        ],
        out_specs=pl.BlockSpec((rows, c), lambda i: (i, 0)),
        out_shape=jax.ShapeDtypeStruct((b, c), jnp.float32),
    )(teacher_logits, lab2)


# speedup vs baseline: 1.0438x; 1.0438x over previous
import jax
import jax.numpy as jnp
from jax import lax
from jax.experimental import pallas as pl

_ALPHA = 0.95


def _row_sum_compensated(x):
    c = x.shape[1]
    partials = [
        jnp.sum(x[:, k : min(k + 128, c)], axis=1, keepdims=True)
        for k in range(0, c, 128)
    ]
    s = partials[0]
    comp = jnp.zeros_like(s)
    for p in partials[1:]:
        t = s + p
        comp = comp + jnp.where(
            jnp.abs(s) >= jnp.abs(p), (s - t) + p, (p - t) + s
        )
        s = t
    return s + comp


def _loca_body(x_ref, lab_ref, out_ref):
    x = x_ref[...]
    lab = lab_ref[...]
    r, c = x.shape
    col = lax.broadcasted_iota(jnp.int32, (r, c), 1)
    mask = col == lab
    rs = _row_sum_compensated(x)
    t = jnp.sum(jnp.where(mask, x, 0.0), axis=1, keepdims=True)
    s = _ALPHA / (1.0 - 2.0 * t + rs)
    tv = 1.0 - s * rs + s * t
    out_ref[...] = jnp.where(mask, tv, s * x)


def kernel(teacher_logits, true_labels):
    b, c = teacher_logits.shape
    rows = 1024
    lab2 = true_labels.astype(jnp.int32).reshape(b, 1)
    return pl.pallas_call(
        _loca_body,
        grid=(b // rows,),
        in_specs=[
            pl.BlockSpec((rows, c), lambda i: (i, 0)),
            pl.BlockSpec((rows, 1), lambda i: (i, 0)),
        ],
        out_specs=pl.BlockSpec((rows, c), lambda i: (i, 0)),
        out_shape=jax.ShapeDtypeStruct((b, c), jnp.float32),
    )(teacher_logits, lab2)


# speedup vs baseline: 1.0595x; 1.0151x over previous
import jax
import jax.numpy as jnp
from jax import lax
from jax.experimental import pallas as pl

_ALPHA = 0.95


def _row_sum_compensated(x):
    c = x.shape[1]
    partials = [
        jnp.sum(x[:, k : min(k + 128, c)], axis=1, keepdims=True)
        for k in range(0, c, 128)
    ]
    s = partials[0]
    comp = jnp.zeros_like(s)
    for p in partials[1:]:
        t = s + p
        comp = comp + jnp.where(
            jnp.abs(s) >= jnp.abs(p), (s - t) + p, (p - t) + s
        )
        s = t
    return s + comp


def _loca_body(x_ref, lab_ref, out_ref):
    x = x_ref[...]
    lab = lab_ref[...]
    r, c = x.shape
    col = lax.broadcasted_iota(jnp.int32, (r, c), 1)
    mask = col == lab
    rs = _row_sum_compensated(x)
    t = jnp.sum(jnp.where(mask, x, 0.0), axis=1, keepdims=True)
    s = _ALPHA / (1.0 - 2.0 * t + rs)
    tv = 1.0 - s * rs + s * t
    out_ref[...] = jnp.where(mask, tv, s * x)


def kernel(teacher_logits, true_labels):
    b, c = teacher_logits.shape
    rows = 2048
    lab2 = true_labels.astype(jnp.int32).reshape(b, 1)
    return pl.pallas_call(
        _loca_body,
        grid=(b // rows,),
        in_specs=[
            pl.BlockSpec((rows, c), lambda i: (i, 0)),
            pl.BlockSpec((rows, 1), lambda i: (i, 0)),
        ],
        out_specs=pl.BlockSpec((rows, c), lambda i: (i, 0)),
        out_shape=jax.ShapeDtypeStruct((b, c), jnp.float32),
    )(teacher_logits, lab2)
